# baseline (speedup 1.0000x reference)
import functools

import jax
import jax.numpy as jnp
from jax import lax
from jax.experimental import pallas as pl
from jax.experimental.pallas import tpu as pltpu
from jax.experimental.pallas import tpu_sc as plsc

B = 16384
F = 26
D = 64
NC, NS = 2, 16
NW = NC * NS
E_PER_W = B // NW
CH = 32
NCHUNK = E_PER_W // CH
GI = 104
G = CH * F // GI
LANES = 16
KD = D // LANES


def _sc_flatten_idx(indices):
    mesh = plsc.VectorSubcoreMesh(core_axis_name="c", subcore_axis_name="s")
    RB = B // NW

    @functools.partial(
        pl.kernel,
        out_type=jax.ShapeDtypeStruct((B * F,), jnp.int32),
        mesh=mesh,
        scratch_types=[
            pltpu.VMEM((RB, F), jnp.int32),
            pltpu.VMEM((RB * F,), jnp.int32),
        ],
        compiler_params=pltpu.CompilerParams(use_tc_tiling_on_sc=True),
    )
    def ka(idx_hbm, out_hbm, idx_a, obuf):
        wid = lax.axis_index("s") * NC + lax.axis_index("c")
        r0 = wid * RB
        pltpu.sync_copy(idx_hbm.at[pl.ds(r0, RB)], idx_a)

        def row_body(r, _):
            v0 = idx_a[r, pl.ds(0, LANES)]
            v1 = idx_a[r, pl.ds(F - LANES, LANES)]
            base = r * F
            obuf[pl.ds(base, LANES)] = v0
            obuf[pl.ds(base + F - LANES, LANES)] = v1
            return 0

        lax.fori_loop(0, RB, row_body, 0)
        pltpu.sync_copy(obuf, out_hbm.at[pl.ds(wid * RB * F, RB * F)])

    return ka(indices)


def _sc_gather_sum(idx1d, table):
    mesh = plsc.VectorSubcoreMesh(core_axis_name="c", subcore_axis_name="s")

    @functools.partial(
        pl.kernel,
        out_type=jax.ShapeDtypeStruct((B, D), jnp.float32),
        mesh=mesh,
        scratch_types=[
            pltpu.VMEM((2, CH * F), jnp.int32),
            pltpu.VMEM((2, CH * F, D), jnp.bfloat16),
            pltpu.VMEM((2, CH, D), jnp.float32),
            pltpu.SemaphoreType.DMA,
            pltpu.SemaphoreType.DMA,
        ],
        compiler_params=pltpu.CompilerParams(use_tc_tiling_on_sc=False,
                                             needs_layout_passes=False),
    )
    def k(idx_hbm, table_hbm, out_hbm, idx_v, rows_v, out_v, sem0, sem1):
        wid = lax.axis_index("s") * NC + lax.axis_index("c")
        out_base = wid * E_PER_W
        sems = (sem0, sem1)

        def issue(c, bslot):
            pltpu.sync_copy(
                idx_hbm.at[pl.ds((out_base + c * CH) * F, CH * F)],
                idx_v.at[bslot])
            pltpu.async_copy(
                table_hbm.at[idx_v.at[bslot]],
                rows_v.at[bslot],
                sems[bslot],
            )

        def drain(bslot):
            pltpu.make_async_copy(
                table_hbm.at[idx_v.at[bslot]],
                rows_v.at[bslot],
                sems[bslot],
            ).wait()

        def tree(vals):
            while len(vals) > 1:
                nxt = [vals[i] + vals[i + 1]
                       for i in range(0, len(vals) - 1, 2)]
                if len(vals) % 2:
                    nxt.append(vals[-1])
                vals = nxt
            return vals[0]

        def accumulate(c, bslot):
            def ent_body(e, _):
                r0 = e * F
                for kk2 in range(D // (2 * LANES)):
                    evens, odds = [], []
                    for f in range(F):
                        x = rows_v[bslot, r0 + f, pl.ds(kk2 * 2 * LANES,
                                                        2 * LANES)]
                        a, bb = plsc.unpack(
                            x, format=plsc.PackFormat.INTERLEAVED)
                        evens.append(a)
                        odds.append(bb)
                    out_v[bslot, e, pl.ds(kk2 * 2 * LANES, LANES)] = tree(evens)
                    out_v[bslot, e,
                          pl.ds(kk2 * 2 * LANES + LANES, LANES)] = tree(odds)
                return 0

            lax.fori_loop(0, CH, ent_body, 0, unroll=4)
            pltpu.sync_copy(out_v.at[bslot],
                            out_hbm.at[pl.ds(out_base + c * CH, CH)])

        issue(0, 0)
        issue(1, 1)

        @pl.loop(0, NCHUNK, step=2)
        def chunk_body(g):
            for bslot in range(2):
                c = g + bslot
                drain(bslot)
                accumulate(c, bslot)

                @pl.when(c + 2 < NCHUNK)
                def _():
                    issue(c + 2, bslot)

    return k(idx1d, table)


def _tc_mlp(summed, W, b):
    BM = 2048

    def body(x_ref, w_ref, b_ref, o_ref):
        y = jnp.dot(x_ref[...], w_ref[...], preferred_element_type=jnp.float32)
        o_ref[...] = jnp.maximum(y + b_ref[...], 0.0)

    return pl.pallas_call(
        body,
        grid=(B // BM,),
        in_specs=[
            pl.BlockSpec((BM, D), lambda i: (i, 0)),
            pl.BlockSpec((D, D), lambda i: (0, 0)),
            pl.BlockSpec((1, D), lambda i: (0, 0)),
        ],
        out_specs=pl.BlockSpec((BM, D), lambda i: (i, 0)),
        out_shape=jax.ShapeDtypeStruct((B, D), jnp.float32),
    )(summed, W, b.reshape(1, D))


def _lane_perm():
    perm = []
    for kk2 in range(D // (2 * LANES)):
        base = kk2 * 2 * LANES
        perm += [base + 2 * j for j in range(LANES)]
        perm += [base + 2 * j + 1 for j in range(LANES)]
    return jnp.array(perm, dtype=jnp.int32)


def kernel(indices, table, W, b):
    idx1d = _sc_flatten_idx(indices)
    summed_p = _sc_gather_sum(idx1d, table.astype(jnp.bfloat16))
    return _tc_mlp(summed_p, W[_lane_perm(), :], b)


# speedup vs baseline: 1.1413x; 1.1413x over previous
import functools

import jax
import jax.numpy as jnp
from jax import lax
from jax.experimental import pallas as pl
from jax.experimental.pallas import tpu as pltpu
from jax.experimental.pallas import tpu_sc as plsc

B = 16384
F = 26
D = 64
NC, NS = 2, 16
NW = NC * NS
E_PER_W = B // NW
CH = 32
NCHUNK = E_PER_W // CH
GI = 104
G = CH * F // GI
LANES = 16
KD = D // LANES


def _sc_flatten_idx(indices):
    mesh = plsc.VectorSubcoreMesh(core_axis_name="c", subcore_axis_name="s")
    RB = B // NW

    @functools.partial(
        pl.kernel,
        out_type=jax.ShapeDtypeStruct((B * F,), jnp.int32),
        mesh=mesh,
        scratch_types=[
            pltpu.VMEM((RB, F), jnp.int32),
            pltpu.VMEM((RB * F,), jnp.int32),
        ],
        compiler_params=pltpu.CompilerParams(use_tc_tiling_on_sc=True),
    )
    def ka(idx_hbm, out_hbm, idx_a, obuf):
        wid = lax.axis_index("s") * NC + lax.axis_index("c")
        r0 = wid * RB
        pltpu.sync_copy(idx_hbm.at[pl.ds(r0, RB)], idx_a)

        def row_body(r, _):
            v0 = idx_a[r, pl.ds(0, LANES)]
            v1 = idx_a[r, pl.ds(F - LANES, LANES)]
            base = r * F
            obuf[pl.ds(base, LANES)] = v0
            obuf[pl.ds(base + F - LANES, LANES)] = v1
            return 0

        lax.fori_loop(0, RB, row_body, 0)
        pltpu.sync_copy(obuf, out_hbm.at[pl.ds(wid * RB * F, RB * F)])

    return ka(indices)


def _sc_gather_sum(idx1d, table):
    mesh = plsc.VectorSubcoreMesh(core_axis_name="c", subcore_axis_name="s")

    @functools.partial(
        pl.kernel,
        out_type=jax.ShapeDtypeStruct((B // 2, 2 * D), jnp.float32),
        mesh=mesh,
        scratch_types=[
            pltpu.VMEM((2, CH * F), jnp.int32),
            pltpu.VMEM((2, CH * F, D), jnp.float32),
            pltpu.VMEM((2, CH // 2, 2 * D), jnp.float32),
            pltpu.SemaphoreType.DMA,
            pltpu.SemaphoreType.DMA,
        ],
        compiler_params=pltpu.CompilerParams(use_tc_tiling_on_sc=False),
    )
    def k(idx_hbm, table_hbm, out_hbm, idx_v, rows_v, out_v, sem0, sem1):
        wid = lax.axis_index("s") * NC + lax.axis_index("c")
        out_base = wid * E_PER_W
        sems = (sem0, sem1)

        def issue(c, bslot):
            pltpu.sync_copy(
                idx_hbm.at[pl.ds((out_base + c * CH) * F, CH * F)],
                idx_v.at[bslot])
            pltpu.async_copy(
                table_hbm.at[idx_v.at[bslot]],
                rows_v.at[bslot],
                sems[bslot],
            )

        def drain(bslot):
            pltpu.make_async_copy(
                table_hbm.at[idx_v.at[bslot]],
                rows_v.at[bslot],
                sems[bslot],
            ).wait()

        def accumulate(c, bslot):
            def ent_body(e, _):
                r0 = e * F
                for kk in range(KD):
                    vals = [rows_v[bslot, r0 + f, pl.ds(kk * LANES, LANES)]
                            for f in range(F)]
                    while len(vals) > 1:
                        nxt = [vals[i] + vals[i + 1]
                               for i in range(0, len(vals) - 1, 2)]
                        if len(vals) % 2:
                            nxt.append(vals[-1])
                        vals = nxt
                    out_v[bslot, e // 2,
                          pl.ds((e % 2) * D + kk * LANES, LANES)] = vals[0]
                return 0

            lax.fori_loop(0, CH, ent_body, 0, unroll=4)
            pltpu.sync_copy(
                out_v.at[bslot],
                out_hbm.at[pl.ds((out_base + c * CH) // 2, CH // 2)])

        issue(0, 0)
        issue(1, 1)

        @pl.loop(0, NCHUNK, step=2)
        def chunk_body(g):
            for bslot in range(2):
                c = g + bslot
                drain(bslot)
                accumulate(c, bslot)

                @pl.when(c + 2 < NCHUNK)
                def _():
                    issue(c + 2, bslot)

    return k(idx1d, table)


def _tc_mlp(summed2, W, b):
    BM = 1024
    W2 = jnp.zeros((2 * D, 2 * D), dtype=W.dtype)
    W2 = W2.at[:D, :D].set(W).at[D:, D:].set(W)
    b2 = jnp.concatenate([b, b]).reshape(1, 2 * D)

    def body(x_ref, w_ref, b_ref, o_ref):
        y = jnp.dot(x_ref[...], w_ref[...], preferred_element_type=jnp.float32)
        o_ref[...] = jnp.maximum(y + b_ref[...], 0.0)

    out2 = pl.pallas_call(
        body,
        grid=(B // 2 // BM,),
        in_specs=[
            pl.BlockSpec((BM, 2 * D), lambda i: (i, 0)),
            pl.BlockSpec((2 * D, 2 * D), lambda i: (0, 0)),
            pl.BlockSpec((1, 2 * D), lambda i: (0, 0)),
        ],
        out_specs=pl.BlockSpec((BM, 2 * D), lambda i: (i, 0)),
        out_shape=jax.ShapeDtypeStruct((B // 2, 2 * D), jnp.float32),
    )(summed2, W2, b2)
    return out2.reshape(B, D)


def kernel(indices, table, W, b):
    idx1d = _sc_flatten_idx(indices)
    summed2 = _sc_gather_sum(idx1d, table)
    return _tc_mlp(summed2, W, b)


# speedup vs baseline: 1.2051x; 1.0559x over previous
import functools

import jax
import jax.numpy as jnp
from jax import lax
from jax.experimental import pallas as pl
from jax.experimental.pallas import tpu as pltpu
from jax.experimental.pallas import tpu_sc as plsc

B = 16384
F = 26
D = 64
NC, NS = 2, 16
NW = NC * NS
E_PER_W = B // NW
CH = 32
NCHUNK = E_PER_W // CH
GI = 104
G = CH * F // GI
LANES = 16
KD = D // LANES


def _sc_flatten_idx(indices):
    mesh = plsc.VectorSubcoreMesh(core_axis_name="c", subcore_axis_name="s")
    RB = B // NW

    @functools.partial(
        pl.kernel,
        out_type=jax.ShapeDtypeStruct((B * F,), jnp.int32),
        mesh=mesh,
        scratch_types=[
            pltpu.VMEM((RB, F), jnp.int32),
            pltpu.VMEM((RB * F,), jnp.int32),
        ],
        compiler_params=pltpu.CompilerParams(use_tc_tiling_on_sc=True),
    )
    def ka(idx_hbm, out_hbm, idx_a, obuf):
        wid = lax.axis_index("s") * NC + lax.axis_index("c")
        r0 = wid * RB
        pltpu.sync_copy(idx_hbm.at[pl.ds(r0, RB)], idx_a)

        def row_body(r, _):
            v0 = idx_a[r, pl.ds(0, LANES)]
            v1 = idx_a[r, pl.ds(F - LANES, LANES)]
            base = r * F
            obuf[pl.ds(base, LANES)] = v0
            obuf[pl.ds(base + F - LANES, LANES)] = v1
            return 0

        lax.fori_loop(0, RB, row_body, 0)
        pltpu.sync_copy(obuf, out_hbm.at[pl.ds(wid * RB * F, RB * F)])

    return ka(indices)


def _sc_gather_sum(idx1d, table):
    mesh = plsc.VectorSubcoreMesh(core_axis_name="c", subcore_axis_name="s")

    @functools.partial(
        pl.kernel,
        out_type=jax.ShapeDtypeStruct((B // 2, 2 * D), jnp.float32),
        mesh=mesh,
        scratch_types=[
            pltpu.VMEM((E_PER_W * F,), jnp.int32),
            pltpu.VMEM((2, CH * F, D), jnp.float32),
            pltpu.VMEM((2, CH // 2, 2 * D), jnp.float32),
            pltpu.SemaphoreType.DMA,
            pltpu.SemaphoreType.DMA,
        ],
        compiler_params=pltpu.CompilerParams(use_tc_tiling_on_sc=False),
    )
    def k(idx_hbm, table_hbm, out_hbm, idx_v, rows_v, out_v, sem0, sem1):
        wid = lax.axis_index("s") * NC + lax.axis_index("c")
        out_base = wid * E_PER_W
        sems = (sem0, sem1)

        pltpu.sync_copy(idx_hbm.at[pl.ds(out_base * F, E_PER_W * F)], idx_v)

        def issue(c, bslot):
            pltpu.async_copy(
                table_hbm.at[idx_v.at[pl.ds(c * CH * F, CH * F)]],
                rows_v.at[bslot],
                sems[bslot],
            )

        def drain(c, bslot):
            pltpu.make_async_copy(
                table_hbm.at[idx_v.at[pl.ds(c * CH * F, CH * F)]],
                rows_v.at[bslot],
                sems[bslot],
            ).wait()

        def accumulate(c, bslot):
            def ent_body(e, _):
                r0 = e * F
                for kk in range(KD):
                    vals = [rows_v[bslot, r0 + f, pl.ds(kk * LANES, LANES)]
                            for f in range(F)]
                    while len(vals) > 1:
                        nxt = [vals[i] + vals[i + 1]
                               for i in range(0, len(vals) - 1, 2)]
                        if len(vals) % 2:
                            nxt.append(vals[-1])
                        vals = nxt
                    out_v[bslot, e // 2,
                          pl.ds((e % 2) * D + kk * LANES, LANES)] = vals[0]
                return 0

            lax.fori_loop(0, CH, ent_body, 0, unroll=4)
            pltpu.sync_copy(
                out_v.at[bslot],
                out_hbm.at[pl.ds((out_base + c * CH) // 2, CH // 2)])

        issue(0, 0)
        issue(1, 1)

        @pl.loop(0, NCHUNK, step=2)
        def chunk_body(g):
            for bslot in range(2):
                c = g + bslot
                drain(c, bslot)
                accumulate(c, bslot)

                @pl.when(c + 2 < NCHUNK)
                def _():
                    issue(c + 2, bslot)

    return k(idx1d, table)


def _tc_mlp(summed2, W, b):
    BM = 1024
    W2 = jnp.zeros((2 * D, 2 * D), dtype=W.dtype)
    W2 = W2.at[:D, :D].set(W).at[D:, D:].set(W)
    b2 = jnp.concatenate([b, b]).reshape(1, 2 * D)

    def body(x_ref, w_ref, b_ref, o_ref):
        y = jnp.dot(x_ref[...], w_ref[...], preferred_element_type=jnp.float32)
        o_ref[...] = jnp.maximum(y + b_ref[...], 0.0)

    out2 = pl.pallas_call(
        body,
        grid=(B // 2 // BM,),
        in_specs=[
            pl.BlockSpec((BM, 2 * D), lambda i: (i, 0)),
            pl.BlockSpec((2 * D, 2 * D), lambda i: (0, 0)),
            pl.BlockSpec((1, 2 * D), lambda i: (0, 0)),
        ],
        out_specs=pl.BlockSpec((BM, 2 * D), lambda i: (i, 0)),
        out_shape=jax.ShapeDtypeStruct((B // 2, 2 * D), jnp.float32),
    )(summed2, W2, b2)
    return out2.reshape(B, D)


def kernel(indices, table, W, b):
    idx1d = _sc_flatten_idx(indices)
    summed2 = _sc_gather_sum(idx1d, table)
    return _tc_mlp(summed2, W, b)
